# baseline (speedup 1.0000x reference)
import functools

import jax
import jax.numpy as jnp
from jax import lax
from jax.experimental import pallas as pl
from jax.experimental.pallas import tpu as pltpu
from jax.experimental.pallas import tpu_sc as plsc

_E = 8
_D = 1024
_F = 2048
_O = 1024
_N = 4096

_TB = 1024
_NB = _N // _TB
_MB = 128
_NT = _N // _MB
_NW = _NT + _E - 1
_WPAD = 64

_NWORK = 32
_RPW = _N // _NWORK
_CH = 64


def _router_body(x_ref, Wr_ref, br_ref,
                 probs_ref, ranks_ref, counts_ref, xp_ref):
    b = pl.program_id(0)
    xb = x_ref[...]

    lo_bits = lax.bitcast_convert_type(
        xb[:, :_D // 2].astype(jnp.bfloat16).astype(jnp.float32), jnp.int32)
    hi_bits = lax.bitcast_convert_type(
        xb[:, _D // 2:].astype(jnp.bfloat16).astype(jnp.float32), jnp.int32)
    xp_ref[...] = ((hi_bits & jnp.int32(-65536))
                   | lax.shift_right_logical(lo_bits, 16))

    logits = jnp.dot(xb, Wr_ref[...], preferred_element_type=jnp.float32)
    logits = logits + br_ref[...][None, :]
    m = jnp.max(logits, axis=-1, keepdims=True)
    unn = jnp.exp(logits - m)
    probs = unn / jnp.sum(unn, axis=-1, keepdims=True)
    probs_ref[...] = probs

    routes = jnp.argmax(probs, axis=-1)
    iota_e = lax.broadcasted_iota(jnp.int32, (1, _E), 1)
    onehot = (routes[:, None] == iota_e).astype(jnp.float32)

    r = lax.broadcasted_iota(jnp.int32, (_TB, _TB), 0)
    c = lax.broadcasted_iota(jnp.int32, (_TB, _TB), 1)
    tril = (c < r).astype(jnp.float32)
    ranks_in = jnp.dot(tril, onehot, preferred_element_type=jnp.float32)

    prev = jnp.where(b == 0, 0.0, counts_ref[...])
    ranks_ref[...] = ranks_in + prev
    counts_ref[...] = prev + jnp.sum(onehot, axis=0, keepdims=True)


def _col_prefix(row, strict):
    a = jnp.broadcast_to(row, (_E, _E))
    r = lax.broadcasted_iota(jnp.int32, (_E, _E), 0)
    c = lax.broadcasted_iota(jnp.int32, (_E, _E), 1)
    sel = (c < r) if strict else (c <= r)
    return jnp.sum(jnp.where(sel, a, 0), axis=1, keepdims=True)


def _row_from_col(col):
    a = jnp.broadcast_to(col, (_E, _E))
    r = lax.broadcasted_iota(jnp.int32, (_E, _E), 0)
    c = lax.broadcasted_iota(jnp.int32, (_E, _E), 1)
    return jnp.sum(jnp.where(r == c, a, 0), axis=0, keepdims=True)


def _slots_body(probs_ref, ranks_ref, counts_ref, slots_ref, meta_ref):
    b = pl.program_id(0)
    counts_row = counts_ref[...]

    off_start_col = _col_prefix(counts_row, strict=True)
    off_start_row = _row_from_col(off_start_col)

    probs = probs_ref[...]
    routes = jnp.argmax(probs, axis=-1)
    iota_e = lax.broadcasted_iota(jnp.int32, (1, _E), 1)
    onehot = (routes[:, None] == iota_e).astype(jnp.float32)
    slot = jnp.sum(onehot * (off_start_row + ranks_ref[...]),
                   axis=1, keepdims=True)
    slots_ref[...] = slot.astype(jnp.int32)

    @pl.when(b == 0)
    def _meta():
        cnt_col = _col_prefix(counts_row, strict=False) - off_start_col
        cnt_i = cnt_col.astype(jnp.int32)
        os_i = off_start_col.astype(jnp.int32)
        oe_i = os_i + cnt_i
        ft = os_i // _MB
        lte = (oe_i + (_MB - 1)) // _MB
        n_tiles = jnp.where(cnt_i > 0, lte - ft, 0)

        nt_row = _row_from_col(n_tiles)
        item_start = _col_prefix(nt_row, strict=True)
        item_end = item_start + n_tiles
        n_items = jnp.sum(n_tiles)

        w = lax.broadcasted_iota(jnp.int32, (1, _WPAD), 1)
        m8 = (item_start <= w) & (w < item_end)
        e_iota = lax.broadcasted_iota(jnp.int32, (_E, _WPAD), 0)

        def sel(col):
            return jnp.sum(jnp.where(m8, jnp.broadcast_to(col, (_E, _WPAD)),
                                     0), axis=0, keepdims=True)

        e_of_w = jnp.sum(jnp.where(m8, e_iota, 0), axis=0, keepdims=True)
        j_of_w = w - sel(item_start)
        t_of_w = sel(ft) + j_of_w
        lo_w = jnp.maximum(sel(os_i), t_of_w * _MB)
        hi_w = jnp.minimum(sel(oe_i), (t_of_w + 1) * _MB)

        valid = w < n_items
        last_e = jnp.max(jnp.where(valid, e_of_w, -1))
        wi_tile = jnp.where(valid, t_of_w, _NT - 1)
        wi_expert = jnp.where(valid, e_of_w, last_e)
        wi_lo = jnp.where(valid, lo_w, _N)
        wi_hi = jnp.where(valid, hi_w, _N)

        r32 = lax.broadcasted_iota(jnp.int32, (_WPAD, _WPAD), 0)
        c32 = lax.broadcasted_iota(jnp.int32, (_WPAD, _WPAD), 1)
        e_bcast = jnp.broadcast_to(wi_expert, (_WPAD, _WPAD))
        e_col = jnp.sum(jnp.where(r32 == c32, e_bcast, 0),
                        axis=1, keepdims=True)
        prev_e = jnp.sum(jnp.where(r32 == c32 - 1,
                                   jnp.broadcast_to(e_col, (_WPAD, _WPAD)),
                                   0), axis=0, keepdims=True)
        wi_first = jnp.where(
            w == 0, 1, (wi_expert != prev_e).astype(jnp.int32))
        f_bcast = jnp.broadcast_to(wi_first, (_WPAD, _WPAD))
        ndist = jnp.sum(jnp.where(c32 <= r32, f_bcast, 0),
                        axis=1, keepdims=True)
        ndist_row = jnp.sum(jnp.where(r32 == c32,
                                      jnp.broadcast_to(ndist,
                                                       (_WPAD, _WPAD)),
                                      0), axis=0, keepdims=True)
        wi_buf = (ndist_row - 1) % 2
        big = jnp.int32(10 ** 6)
        cand = jnp.where(
            jnp.broadcast_to(e_col, (_WPAD, _WPAD)) > e_bcast,
            jnp.broadcast_to(e_col, (_WPAD, _WPAD)), big)
        wi_next = jnp.min(cand, axis=0, keepdims=True)
        wi_next = jnp.where(wi_next >= big, -1, wi_next)

        meta_ref[...] = jnp.concatenate(
            [wi_tile, wi_expert, wi_lo, wi_hi,
             wi_buf, wi_first, wi_next,
             jnp.zeros((1, _WPAD), jnp.int32)], axis=0)


def _mlp_body(meta_ref, xs_ref, W1_hbm, b1_ref, W2_hbm, b2_ref, ys_ref,
              w1v, w2v, sems):
    w = pl.program_id(0)
    tile = meta_ref[0, w]
    e = meta_ref[1, w]
    lo = meta_ref[2, w]
    hi = meta_ref[3, w]
    buf = meta_ref[4, w]
    first = meta_ref[5, w]
    nxt = meta_ref[6, w]

    def w_copies(expert, b):
        return (pltpu.make_async_copy(W1_hbm.at[expert], w1v.at[b],
                                      sems.at[b]),
                pltpu.make_async_copy(W2_hbm.at[expert], w2v.at[b],
                                      sems.at[b]))

    @pl.when(w == 0)
    def _boot():
        c1, c2 = w_copies(e, buf)
        c1.start()
        c2.start()

    @pl.when(first == 1)
    def _turnover():
        @pl.when(nxt >= 0)
        def _prefetch():
            c1, c2 = w_copies(nxt, 1 - buf)
            c1.start()
            c2.start()

        c1, c2 = w_copies(e, buf)
        c1.wait()
        c2.wait()

    rows = tile * _MB + lax.broadcasted_iota(jnp.int32, (_MB, 1), 0)
    mask = (rows >= lo) & (rows < hi)

    packed = xs_ref[...]
    lo_f = lax.bitcast_convert_type(
        lax.shift_left(packed, 16), jnp.float32)
    hi_f = lax.bitcast_convert_type(
        packed & jnp.int32(-65536), jnp.float32)
    xb = jnp.concatenate([lo_f, hi_f], axis=1).astype(jnp.bfloat16)
    h = jnp.maximum(
        jnp.dot(xb, w1v[buf].astype(jnp.bfloat16),
                preferred_element_type=jnp.float32)
        + b1_ref[0, 0][None, :], 0.0).astype(jnp.bfloat16)
    y = jnp.dot(h, w2v[buf].astype(jnp.bfloat16),
                preferred_element_type=jnp.float32) \
        + b2_ref[0, 0][None, :]

    t_prev = jnp.where(w == 0, -1, meta_ref[0, jnp.maximum(w - 1, 0)])
    prev = jnp.where(tile != t_prev, 0.0, ys_ref[...])
    ys_ref[...] = jnp.where(mask, y, prev)


def _sc_mesh():
    return plsc.VectorSubcoreMesh(core_axis_name="c", subcore_axis_name="s")


def _scatter_rows(x, slots):
    @functools.partial(
        pl.kernel, mesh=_sc_mesh(),
        out_type=jax.ShapeDtypeStruct((_N, _D // 2), jnp.int32),
        scratch_types=[
            pltpu.VMEM((_RPW,), jnp.int32),
            pltpu.VMEM((_RPW, _D // 2), jnp.int32),
            pltpu.SemaphoreType.DMA,
        ],
    )
    def k(x_hbm, slots_hbm, out_hbm, idx_v, rows_v, sem):
        wid = lax.axis_index("s") * 2 + lax.axis_index("c")
        base = wid * _RPW
        pltpu.sync_copy(slots_hbm.at[pl.ds(base, _RPW)], idx_v)
        pltpu.sync_copy(x_hbm.at[pl.ds(base, _RPW)], rows_v)
        pltpu.async_copy(rows_v, out_hbm.at[idx_v], sem).wait()

    return k(x, slots)


def _gather_rows(ys, slots):
    @functools.partial(
        pl.kernel, mesh=_sc_mesh(),
        out_type=jax.ShapeDtypeStruct((_N, _O), jnp.float32),
        scratch_types=[
            pltpu.VMEM((_CH,), jnp.int32),
            pltpu.VMEM((_CH, _O), jnp.float32),
            pltpu.SemaphoreType.DMA,
        ],
    )
    def k(ys_hbm, slots_hbm, out_hbm, idx_v, rows_v, sem):
        wid = lax.axis_index("s") * 2 + lax.axis_index("c")
        base = wid * _RPW
        for ci in range(_RPW // _CH):
            off = base + ci * _CH
            pltpu.sync_copy(slots_hbm.at[pl.ds(off, _CH)], idx_v)
            pltpu.async_copy(ys_hbm.at[idx_v], rows_v, sem).wait()
            pltpu.sync_copy(rows_v, out_hbm.at[pl.ds(off, _CH)])

    return k(ys, slots)


def kernel(x, Wr, br, W1, b1, W2, b2):
    probs, ranks, counts, xp = pl.pallas_call(
        _router_body,
        grid=(_NB,),
        in_specs=[
            pl.BlockSpec((_TB, _D), lambda b: (b, 0)),
            pl.BlockSpec((_D, _E), lambda b: (0, 0)),
            pl.BlockSpec((_E,), lambda b: (0,)),
        ],
        out_specs=[
            pl.BlockSpec((_TB, _E), lambda b: (b, 0)),
            pl.BlockSpec((_TB, _E), lambda b: (b, 0)),
            pl.BlockSpec((1, _E), lambda b: (0, 0)),
            pl.BlockSpec((_TB, _D // 2), lambda b: (b, 0)),
        ],
        out_shape=[
            jax.ShapeDtypeStruct((_N, _E), jnp.float32),
            jax.ShapeDtypeStruct((_N, _E), jnp.float32),
            jax.ShapeDtypeStruct((1, _E), jnp.float32),
            jax.ShapeDtypeStruct((_N, _D // 2), jnp.int32),
        ],
        compiler_params=pltpu.CompilerParams(
            dimension_semantics=("arbitrary",)),
    )(x, Wr, br)

    slots2d, meta = pl.pallas_call(
        _slots_body,
        grid=(_NB,),
        in_specs=[
            pl.BlockSpec((_TB, _E), lambda b: (b, 0)),
            pl.BlockSpec((_TB, _E), lambda b: (b, 0)),
            pl.BlockSpec((1, _E), lambda b: (0, 0)),
        ],
        out_specs=[
            pl.BlockSpec((_TB, 1), lambda b: (b, 0)),
            pl.BlockSpec((8, _WPAD), lambda b: (0, 0)),
        ],
        out_shape=[
            jax.ShapeDtypeStruct((_N, 1), jnp.int32),
            jax.ShapeDtypeStruct((8, _WPAD), jnp.int32),
        ],
        compiler_params=pltpu.CompilerParams(
            dimension_semantics=("arbitrary",)),
    )(probs, ranks, counts)

    slots = slots2d.reshape(_N)
    xs = _scatter_rows(xp, slots)

    grid_spec = pltpu.PrefetchScalarGridSpec(
        num_scalar_prefetch=1,
        grid=(_NW,),
        in_specs=[
            pl.BlockSpec((_MB, _D // 2), lambda w, m: (m[0, w], 0)),
            pl.BlockSpec(memory_space=pl.ANY),
            pl.BlockSpec((1, 1, _F), lambda w, m: (m[1, w], 0, 0)),
            pl.BlockSpec(memory_space=pl.ANY),
            pl.BlockSpec((1, 1, _O), lambda w, m: (m[1, w], 0, 0)),
        ],
        out_specs=pl.BlockSpec((_MB, _O), lambda w, m: (m[0, w], 0)),
        scratch_shapes=[
            pltpu.VMEM((2, _D, _F), jnp.float32),
            pltpu.VMEM((2, _F, _O), jnp.float32),
            pltpu.SemaphoreType.DMA((2,)),
        ],
    )
    ys = pl.pallas_call(
        _mlp_body,
        grid_spec=grid_spec,
        out_shape=jax.ShapeDtypeStruct((_N, _O), jnp.float32),
        compiler_params=pltpu.CompilerParams(
            dimension_semantics=("arbitrary",)),
    )(meta, xs, W1, b1.reshape(_E, 1, _F), W2, b2.reshape(_E, 1, _O))

    out = _gather_rows(ys, slots)
    return out, probs, counts.reshape(_E)


# speedup vs baseline: 1.0791x; 1.0791x over previous
import functools

import jax
import jax.numpy as jnp
from jax import lax
from jax.experimental import pallas as pl
from jax.experimental.pallas import tpu as pltpu
from jax.experimental.pallas import tpu_sc as plsc

_E = 8
_D = 1024
_F = 2048
_O = 1024
_N = 4096

_TB = 1024
_NB = _N // _TB
_MB = 256
_NT = _N // _MB
_NW = _NT + _E - 1
_WPAD = 32

_NWORK = 32
_RPW = _N // _NWORK
_CH = 64


def _router_body(x_ref, Wr_ref, br_ref,
                 probs_ref, ranks_ref, counts_ref, xp_ref):
    b = pl.program_id(0)
    xb = x_ref[...]

    lo_bits = lax.bitcast_convert_type(
        xb[:, :_D // 2].astype(jnp.bfloat16).astype(jnp.float32), jnp.int32)
    hi_bits = lax.bitcast_convert_type(
        xb[:, _D // 2:].astype(jnp.bfloat16).astype(jnp.float32), jnp.int32)
    xp_ref[...] = ((hi_bits & jnp.int32(-65536))
                   | lax.shift_right_logical(lo_bits, 16))

    logits = jnp.dot(xb, Wr_ref[...], preferred_element_type=jnp.float32)
    logits = logits + br_ref[...][None, :]
    m = jnp.max(logits, axis=-1, keepdims=True)
    unn = jnp.exp(logits - m)
    probs = unn / jnp.sum(unn, axis=-1, keepdims=True)
    probs_ref[...] = probs

    routes = jnp.argmax(probs, axis=-1)
    iota_e = lax.broadcasted_iota(jnp.int32, (1, _E), 1)
    onehot = (routes[:, None] == iota_e).astype(jnp.float32)

    r = lax.broadcasted_iota(jnp.int32, (_TB, _TB), 0)
    c = lax.broadcasted_iota(jnp.int32, (_TB, _TB), 1)
    tril = (c < r).astype(jnp.float32)
    ranks_in = jnp.dot(tril, onehot, preferred_element_type=jnp.float32)

    prev = jnp.where(b == 0, 0.0, counts_ref[...])
    ranks_ref[...] = ranks_in + prev
    counts_ref[...] = prev + jnp.sum(onehot, axis=0, keepdims=True)


def _col_prefix(row, strict):
    a = jnp.broadcast_to(row, (_E, _E))
    r = lax.broadcasted_iota(jnp.int32, (_E, _E), 0)
    c = lax.broadcasted_iota(jnp.int32, (_E, _E), 1)
    sel = (c < r) if strict else (c <= r)
    return jnp.sum(jnp.where(sel, a, 0), axis=1, keepdims=True)


def _row_from_col(col):
    a = jnp.broadcast_to(col, (_E, _E))
    r = lax.broadcasted_iota(jnp.int32, (_E, _E), 0)
    c = lax.broadcasted_iota(jnp.int32, (_E, _E), 1)
    return jnp.sum(jnp.where(r == c, a, 0), axis=0, keepdims=True)


def _slots_body(probs_ref, ranks_ref, counts_ref, slots_ref, meta_ref):
    b = pl.program_id(0)
    counts_row = counts_ref[...]

    off_start_col = _col_prefix(counts_row, strict=True)
    off_start_row = _row_from_col(off_start_col)

    probs = probs_ref[...]
    routes = jnp.argmax(probs, axis=-1)
    iota_e = lax.broadcasted_iota(jnp.int32, (1, _E), 1)
    onehot = (routes[:, None] == iota_e).astype(jnp.float32)
    slot = jnp.sum(onehot * (off_start_row + ranks_ref[...]),
                   axis=1, keepdims=True)
    slots_ref[...] = slot.astype(jnp.int32)

    @pl.when(b == 0)
    def _meta():
        cnt_col = _col_prefix(counts_row, strict=False) - off_start_col
        cnt_i = cnt_col.astype(jnp.int32)
        os_i = off_start_col.astype(jnp.int32)
        oe_i = os_i + cnt_i
        ft = os_i // _MB
        lte = (oe_i + (_MB - 1)) // _MB
        n_tiles = jnp.where(cnt_i > 0, lte - ft, 0)

        nt_row = _row_from_col(n_tiles)
        item_start = _col_prefix(nt_row, strict=True)
        item_end = item_start + n_tiles
        n_items = jnp.sum(n_tiles)

        w = lax.broadcasted_iota(jnp.int32, (1, _WPAD), 1)
        m8 = (item_start <= w) & (w < item_end)
        e_iota = lax.broadcasted_iota(jnp.int32, (_E, _WPAD), 0)

        def sel(col):
            return jnp.sum(jnp.where(m8, jnp.broadcast_to(col, (_E, _WPAD)),
                                     0), axis=0, keepdims=True)

        e_of_w = jnp.sum(jnp.where(m8, e_iota, 0), axis=0, keepdims=True)
        j_of_w = w - sel(item_start)
        t_of_w = sel(ft) + j_of_w
        lo_w = jnp.maximum(sel(os_i), t_of_w * _MB)
        hi_w = jnp.minimum(sel(oe_i), (t_of_w + 1) * _MB)

        valid = w < n_items
        last_e = jnp.max(jnp.where(valid, e_of_w, -1))
        wi_tile = jnp.where(valid, t_of_w, _NT - 1)
        wi_expert = jnp.where(valid, e_of_w, last_e)
        wi_lo = jnp.where(valid, lo_w, _N)
        wi_hi = jnp.where(valid, hi_w, _N)

        r32 = lax.broadcasted_iota(jnp.int32, (_WPAD, _WPAD), 0)
        c32 = lax.broadcasted_iota(jnp.int32, (_WPAD, _WPAD), 1)
        e_bcast = jnp.broadcast_to(wi_expert, (_WPAD, _WPAD))
        e_col = jnp.sum(jnp.where(r32 == c32, e_bcast, 0),
                        axis=1, keepdims=True)
        prev_e = jnp.sum(jnp.where(r32 == c32 - 1,
                                   jnp.broadcast_to(e_col, (_WPAD, _WPAD)),
                                   0), axis=0, keepdims=True)
        wi_first = jnp.where(
            w == 0, 1, (wi_expert != prev_e).astype(jnp.int32))
        f_bcast = jnp.broadcast_to(wi_first, (_WPAD, _WPAD))
        ndist = jnp.sum(jnp.where(c32 <= r32, f_bcast, 0),
                        axis=1, keepdims=True)
        ndist_row = jnp.sum(jnp.where(r32 == c32,
                                      jnp.broadcast_to(ndist,
                                                       (_WPAD, _WPAD)),
                                      0), axis=0, keepdims=True)
        wi_buf = (ndist_row - 1) % 2
        big = jnp.int32(10 ** 6)
        cand = jnp.where(
            jnp.broadcast_to(e_col, (_WPAD, _WPAD)) > e_bcast,
            jnp.broadcast_to(e_col, (_WPAD, _WPAD)), big)
        wi_next = jnp.min(cand, axis=0, keepdims=True)
        wi_next = jnp.where(wi_next >= big, -1, wi_next)

        meta_ref[...] = jnp.concatenate(
            [wi_tile, wi_expert, wi_lo, wi_hi,
             wi_buf, wi_first, wi_next,
             jnp.zeros((1, _WPAD), jnp.int32)], axis=0)


def _mlp_body(meta_ref, xs_ref, W1_hbm, b1_ref, W2_hbm, b2_ref, ys_ref,
              w1v, w2v, sems):
    w = pl.program_id(0)
    tile = meta_ref[0, w]
    e = meta_ref[1, w]
    lo = meta_ref[2, w]
    hi = meta_ref[3, w]
    buf = meta_ref[4, w]
    first = meta_ref[5, w]
    nxt = meta_ref[6, w]

    def w_copies(expert, b):
        return (pltpu.make_async_copy(W1_hbm.at[expert], w1v.at[b],
                                      sems.at[b]),
                pltpu.make_async_copy(W2_hbm.at[expert], w2v.at[b],
                                      sems.at[b]))

    @pl.when(w == 0)
    def _boot():
        c1, c2 = w_copies(e, buf)
        c1.start()
        c2.start()

    @pl.when(first == 1)
    def _turnover():
        @pl.when(nxt >= 0)
        def _prefetch():
            c1, c2 = w_copies(nxt, 1 - buf)
            c1.start()
            c2.start()

        c1, c2 = w_copies(e, buf)
        c1.wait()
        c2.wait()

    rows = tile * _MB + lax.broadcasted_iota(jnp.int32, (_MB, 1), 0)
    mask = (rows >= lo) & (rows < hi)

    packed = xs_ref[...]
    lo_f = lax.bitcast_convert_type(
        lax.shift_left(packed, 16), jnp.float32)
    hi_f = lax.bitcast_convert_type(
        packed & jnp.int32(-65536), jnp.float32)
    xb = jnp.concatenate([lo_f, hi_f], axis=1).astype(jnp.bfloat16)
    h = jnp.maximum(
        jnp.dot(xb, w1v[buf].astype(jnp.bfloat16),
                preferred_element_type=jnp.float32)
        + b1_ref[0, 0][None, :], 0.0).astype(jnp.bfloat16)
    y = jnp.dot(h, w2v[buf].astype(jnp.bfloat16),
                preferred_element_type=jnp.float32) \
        + b2_ref[0, 0][None, :]

    t_prev = jnp.where(w == 0, -1, meta_ref[0, jnp.maximum(w - 1, 0)])
    prev = jnp.where(tile != t_prev, 0.0, ys_ref[...])
    ys_ref[...] = jnp.where(mask, y, prev)


def _sc_mesh():
    return plsc.VectorSubcoreMesh(core_axis_name="c", subcore_axis_name="s")


def _scatter_rows(x, slots):
    @functools.partial(
        pl.kernel, mesh=_sc_mesh(),
        out_type=jax.ShapeDtypeStruct((_N, _D // 2), jnp.int32),
        scratch_types=[
            pltpu.VMEM((_RPW,), jnp.int32),
            pltpu.VMEM((_RPW, _D // 2), jnp.int32),
            pltpu.SemaphoreType.DMA,
        ],
    )
    def k(x_hbm, slots_hbm, out_hbm, idx_v, rows_v, sem):
        wid = lax.axis_index("s") * 2 + lax.axis_index("c")
        base = wid * _RPW
        pltpu.sync_copy(slots_hbm.at[pl.ds(base, _RPW)], idx_v)
        pltpu.sync_copy(x_hbm.at[pl.ds(base, _RPW)], rows_v)
        pltpu.async_copy(rows_v, out_hbm.at[idx_v], sem).wait()

    return k(x, slots)


def _gather_rows(ys, slots):
    @functools.partial(
        pl.kernel, mesh=_sc_mesh(),
        out_type=jax.ShapeDtypeStruct((_N, _O), jnp.float32),
        scratch_types=[
            pltpu.VMEM((_CH,), jnp.int32),
            pltpu.VMEM((_CH, _O), jnp.float32),
            pltpu.SemaphoreType.DMA,
        ],
    )
    def k(ys_hbm, slots_hbm, out_hbm, idx_v, rows_v, sem):
        wid = lax.axis_index("s") * 2 + lax.axis_index("c")
        base = wid * _RPW
        for ci in range(_RPW // _CH):
            off = base + ci * _CH
            pltpu.sync_copy(slots_hbm.at[pl.ds(off, _CH)], idx_v)
            pltpu.async_copy(ys_hbm.at[idx_v], rows_v, sem).wait()
            pltpu.sync_copy(rows_v, out_hbm.at[pl.ds(off, _CH)])

    return k(ys, slots)


def kernel(x, Wr, br, W1, b1, W2, b2):
    probs, ranks, counts, xp = pl.pallas_call(
        _router_body,
        grid=(_NB,),
        in_specs=[
            pl.BlockSpec((_TB, _D), lambda b: (b, 0)),
            pl.BlockSpec((_D, _E), lambda b: (0, 0)),
            pl.BlockSpec((_E,), lambda b: (0,)),
        ],
        out_specs=[
            pl.BlockSpec((_TB, _E), lambda b: (b, 0)),
            pl.BlockSpec((_TB, _E), lambda b: (b, 0)),
            pl.BlockSpec((1, _E), lambda b: (0, 0)),
            pl.BlockSpec((_TB, _D // 2), lambda b: (b, 0)),
        ],
        out_shape=[
            jax.ShapeDtypeStruct((_N, _E), jnp.float32),
            jax.ShapeDtypeStruct((_N, _E), jnp.float32),
            jax.ShapeDtypeStruct((1, _E), jnp.float32),
            jax.ShapeDtypeStruct((_N, _D // 2), jnp.int32),
        ],
        compiler_params=pltpu.CompilerParams(
            dimension_semantics=("arbitrary",)),
    )(x, Wr, br)

    slots2d, meta = pl.pallas_call(
        _slots_body,
        grid=(_NB,),
        in_specs=[
            pl.BlockSpec((_TB, _E), lambda b: (b, 0)),
            pl.BlockSpec((_TB, _E), lambda b: (b, 0)),
            pl.BlockSpec((1, _E), lambda b: (0, 0)),
        ],
        out_specs=[
            pl.BlockSpec((_TB, 1), lambda b: (b, 0)),
            pl.BlockSpec((8, _WPAD), lambda b: (0, 0)),
        ],
        out_shape=[
            jax.ShapeDtypeStruct((_N, 1), jnp.int32),
            jax.ShapeDtypeStruct((8, _WPAD), jnp.int32),
        ],
        compiler_params=pltpu.CompilerParams(
            dimension_semantics=("arbitrary",)),
    )(probs, ranks, counts)

    slots = slots2d.reshape(_N)
    xs = _scatter_rows(xp, slots)

    grid_spec = pltpu.PrefetchScalarGridSpec(
        num_scalar_prefetch=1,
        grid=(_NW,),
        in_specs=[
            pl.BlockSpec((_MB, _D // 2), lambda w, m: (m[0, w], 0)),
            pl.BlockSpec(memory_space=pl.ANY),
            pl.BlockSpec((1, 1, _F), lambda w, m: (m[1, w], 0, 0)),
            pl.BlockSpec(memory_space=pl.ANY),
            pl.BlockSpec((1, 1, _O), lambda w, m: (m[1, w], 0, 0)),
        ],
        out_specs=pl.BlockSpec((_MB, _O), lambda w, m: (m[0, w], 0)),
        scratch_shapes=[
            pltpu.VMEM((2, _D, _F), jnp.float32),
            pltpu.VMEM((2, _F, _O), jnp.float32),
            pltpu.SemaphoreType.DMA((2,)),
        ],
    )
    ys = pl.pallas_call(
        _mlp_body,
        grid_spec=grid_spec,
        out_shape=jax.ShapeDtypeStruct((_N, _O), jnp.float32),
        compiler_params=pltpu.CompilerParams(
            dimension_semantics=("arbitrary",)),
    )(meta, xs, W1, b1.reshape(_E, 1, _F), W2, b2.reshape(_E, 1, _O))

    out = _gather_rows(ys, slots)
    return out, probs, counts.reshape(_E)
